# baseline (speedup 1.0000x reference)
import jax
import jax.numpy as jnp
from jax.experimental import pallas as pl

_BLOCK_N = 8


def _discretize_body(w_ref, o_ref):
    x = jnp.clip(w_ref[...], 0.0, 1.0)
    o_ref[...] = jnp.round(x * 255.0) / 255.0


def kernel(watermark_samples, response):
    n, c, h, w = watermark_samples.shape
    out = pl.pallas_call(
        _discretize_body,
        grid=(n // _BLOCK_N,),
        in_specs=[pl.BlockSpec((_BLOCK_N, c, h, w), lambda i: (i, 0, 0, 0))],
        out_specs=pl.BlockSpec((_BLOCK_N, c, h, w), lambda i: (i, 0, 0, 0)),
        out_shape=jax.ShapeDtypeStruct((n, c, h, w), jnp.float32),
    )(watermark_samples)
    return (out, response)


# speedup vs baseline: 1.0034x; 1.0034x over previous
import jax
import jax.numpy as jnp
from jax.experimental import pallas as pl

_BLOCK_N = 16


def _discretize_body(w_ref, o_ref):
    x = jnp.clip(w_ref[...], 0.0, 1.0)
    o_ref[...] = jnp.round(x * 255.0) / 255.0


def kernel(watermark_samples, response):
    n, c, h, w = watermark_samples.shape
    out = pl.pallas_call(
        _discretize_body,
        grid=(n // _BLOCK_N,),
        in_specs=[pl.BlockSpec((_BLOCK_N, c, h, w), lambda i: (i, 0, 0, 0))],
        out_specs=pl.BlockSpec((_BLOCK_N, c, h, w), lambda i: (i, 0, 0, 0)),
        out_shape=jax.ShapeDtypeStruct((n, c, h, w), jnp.float32),
    )(watermark_samples)
    return (out, response)
